# baseline (speedup 1.0000x reference)
import functools

import jax
import jax.numpy as jnp
from jax import lax
from jax.experimental import pallas as pl
from jax.experimental.pallas import tpu as pltpu
from jax.experimental.pallas import tpu_sc as plsc

VOCAB = 100000
EMBED_DIM = 128
BATCH = 1024
HIST = 200

NUM_CORES = 2
NUM_SUBCORES = 16
NUM_WORKERS = NUM_CORES * NUM_SUBCORES

B_TOTAL = BATCH * HIST
PER_WORKER = B_TOTAL // NUM_WORKERS
CHUNK = 128
N_CHUNKS = PER_WORKER // CHUNK
NBUF = 7
N_MAIN = (N_CHUNKS // NBUF) * NBUF

_mesh = plsc.VectorSubcoreMesh(core_axis_name="c", subcore_axis_name="s")


@functools.partial(
    pl.kernel,
    out_type=jax.ShapeDtypeStruct((B_TOTAL, EMBED_DIM), jnp.float32),
    mesh=_mesh,
    scratch_types=[
        pltpu.VMEM((N_CHUNKS, CHUNK), jnp.int32),
        [pltpu.VMEM((CHUNK, EMBED_DIM), jnp.float32)] * NBUF,
        [pltpu.SemaphoreType.DMA] * NBUF,
        [pltpu.SemaphoreType.DMA] * NBUF,
    ],
)
def _gather_kernel(idx_hbm, table_hbm, out_hbm, idx_v, rows, gsems, osems):
    wid = lax.axis_index("s") * NUM_CORES + lax.axis_index("c")
    base = pl.multiple_of(wid * PER_WORKER, CHUNK)
    pltpu.sync_copy(idx_hbm.at[wid], idx_v)

    def start_gather(j, b):
        pltpu.async_copy(table_hbm.at[idx_v.at[j]], rows[b], gsems[b])

    def wait_gather(j, b):
        pltpu.make_async_copy(
            table_hbm.at[idx_v.at[j]], rows[b], gsems[b]).wait()

    def start_out(j, b):
        off = pl.multiple_of(base + j * CHUNK, CHUNK)
        pltpu.async_copy(rows[b], out_hbm.at[pl.ds(off, CHUNK)], osems[b])

    def wait_out(b):
        pltpu.make_async_copy(
            rows[b], out_hbm.at[pl.ds(0, CHUNK)], osems[b]).wait()

    for k in range(NBUF - 1):
        start_gather(k, k)

    def body(i, carry):
        o = i * NBUF
        for b in range(NBUF):
            j = o + b
            bn = (b - 1) % NBUF

            @pl.when(j >= 1)
            def _wait_prev_out():
                wait_out(bn)

            @pl.when(j + NBUF - 1 < N_CHUNKS)
            def _start_next_gather():
                start_gather(j + NBUF - 1, bn)

            wait_gather(j, b)
            start_out(j, b)
        return carry

    lax.fori_loop(0, N_MAIN // NBUF, body, 0)

    for j in range(N_MAIN, N_CHUNKS):
        wait_out((j - 1) % NBUF)
        wait_gather(j, j % NBUF)
        start_out(j, j % NBUF)

    wait_out((N_CHUNKS - 1) % NBUF)


def kernel(inputs, w):
    idx = inputs.astype(jnp.int32).reshape(NUM_WORKERS, N_CHUNKS, CHUNK)
    out = _gather_kernel(idx, w)
    return out.reshape(BATCH, HIST, EMBED_DIM)
